# baseline (speedup 1.0000x reference)
import jax
import jax.numpy as jnp
from jax import lax
from jax.experimental import pallas as pl
from jax.experimental.pallas import tpu as pltpu
from jax.experimental.pallas import tpu_sc as plsc

N_NODES = 100000
D_FEAT = 128
N_SEG = 512
CHUNK = 128
NFULL = N_NODES // CHUNK
TAIL = N_NODES - NFULL * CHUNK
NC = 2
NS = 16
NW = NC * NS
CPW = (NFULL + NW - 1) // NW
NTRI = (CPW + 2) // 3


def _phase1(x_hbm, b_hbm, part_hbm, cntp_hbm,
            xb0, xb1, xb2, ib0, ib1, ib2, tidx, cnt_local, acc_sh,
            semx0, semi0, sems0, semx1, semi1, sems1, semx2, semi2, sems2):
    cid = lax.axis_index("c")
    sid = lax.axis_index("s")
    w = sid * NC + cid

    def _init_cnt(i, _):
        cnt_local[i, :] = jnp.zeros((16,), jnp.float32)
        return 0
    lax.fori_loop(0, N_SEG, _init_cnt, 0)

    def _init_z(k, _):
        xb0[k // 8, pl.ds((k % 8) * 16, 16)] = jnp.zeros((16,), jnp.float32)
        return 0
    lax.fori_loop(0, 32 * 8, _init_z, 0)
    pltpu.sync_copy(xb0.at[pl.ds(0, 32)], acc_sh.at[pl.ds(sid * 32, 32)])

    def start_in(g, xb, ib, semx, semi):
        row0 = g * CHUNK
        pltpu.async_copy(b_hbm.at[pl.ds(row0, CHUNK)], ib, semi)
        pltpu.async_copy(x_hbm.at[pl.ds(row0, CHUNK)], xb, semx)

    def wait_in(xb, ib, semx, semi):
        pltpu.make_async_copy(b_hbm.at[pl.ds(0, CHUNK)], ib, semi).wait()
        pltpu.make_async_copy(x_hbm.at[pl.ds(0, CHUNK)], xb, semx).wait()

    ones16 = jnp.ones((16,), jnp.float32)

    def count_chunk(ib, ngroups):
        def body(j, _):
            v = ib[pl.ds(j * 16, 16)]
            for k in range(16):
                plsc.addupdate(cnt_local.at[v[k]], ones16)
            return 0
        lax.fori_loop(0, ngroups, body, 0)

    start_in(w, xb0, ib0, semx0, semi0)
    start_in(w + NW, xb1, ib1, semx1, semi1)
    plsc.subcore_barrier()

    def _tri(c3, _):
        k0 = 3 * c3
        g0 = w + NW * k0
        g1 = g0 + NW
        g2 = g1 + NW
        g2p = g2 - 3 * NW
        g0n = g0 + 3 * NW
        g1n = g1 + 3 * NW

        @pl.when(jnp.logical_and(c3 > 0, g2p < NFULL))
        def _():
            pltpu.make_async_copy(xb2, acc_sh.at[ib2], sems2).wait()

        @pl.when(g2 < NFULL)
        def _():
            start_in(g2, xb2, ib2, semx2, semi2)

        @pl.when(g0 < NFULL)
        def _():
            wait_in(xb0, ib0, semx0, semi0)
            pltpu.async_copy(xb0, acc_sh.at[ib0], sems0, add=True)
            count_chunk(ib0, CHUNK // 16)

        @pl.when(g1 < NFULL)
        def _():
            wait_in(xb1, ib1, semx1, semi1)
            pltpu.async_copy(xb1, acc_sh.at[ib1], sems1, add=True)
            count_chunk(ib1, CHUNK // 16)

        @pl.when(g2 < NFULL)
        def _():
            wait_in(xb2, ib2, semx2, semi2)
            pltpu.async_copy(xb2, acc_sh.at[ib2], sems2, add=True)
            count_chunk(ib2, CHUNK // 16)

        @pl.when(g0 < NFULL)
        def _():
            pltpu.make_async_copy(xb0, acc_sh.at[ib0], sems0).wait()

        @pl.when(g0n < NFULL)
        def _():
            start_in(g0n, xb0, ib0, semx0, semi0)

        @pl.when(g1 < NFULL)
        def _():
            pltpu.make_async_copy(xb1, acc_sh.at[ib1], sems1).wait()

        @pl.when(g1n < NFULL)
        def _():
            start_in(g1n, xb1, ib1, semx1, semi1)
        return 0

    lax.fori_loop(0, NTRI, _tri, 0)

    last_k2 = 3 * (NTRI - 1) + 2
    @pl.when(w + NW * last_k2 < NFULL)
    def _():
        pltpu.make_async_copy(xb2, acc_sh.at[ib2], sems2).wait()

    @pl.when(w == (NFULL % NW))
    def _():
        row0 = NFULL * CHUNK
        pltpu.async_copy(b_hbm.at[pl.ds(row0, TAIL)], tidx, semi0).wait()
        pltpu.async_copy(x_hbm.at[pl.ds(row0, TAIL)],
                         xb0.at[pl.ds(0, TAIL)], semx0).wait()
        pltpu.async_copy(xb0.at[pl.ds(0, TAIL)], acc_sh.at[tidx],
                         sems0, add=True)
        count_chunk(tidx, TAIL // 16)
        pltpu.make_async_copy(xb0.at[pl.ds(0, TAIL)], acc_sh.at[tidx],
                              sems0).wait()

    pltpu.sync_copy(cnt_local, cntp_hbm.at[w])

    plsc.subcore_barrier()

    @pl.when(sid == 0)
    def _():
        pltpu.sync_copy(acc_sh, part_hbm.at[cid])


def _combine(p_ref, c_ref, o_ref):
    s = p_ref[0] + p_ref[1]
    cnt = jnp.sum(c_ref[...], axis=0)[:, 0:1]
    o_ref[...] = s / jnp.maximum(cnt, 1.0)


@jax.jit
def kernel(x, batch):
    mesh = plsc.VectorSubcoreMesh(core_axis_name="c", subcore_axis_name="s")
    phase1 = pl.kernel(
        _phase1,
        out_type=[
            jax.ShapeDtypeStruct((NC, N_SEG, D_FEAT), jnp.float32),
            jax.ShapeDtypeStruct((NW, N_SEG, 16), jnp.float32),
        ],
        mesh=mesh,
        scratch_types=[
            pltpu.VMEM((CHUNK, D_FEAT), jnp.float32),
            pltpu.VMEM((CHUNK, D_FEAT), jnp.float32),
            pltpu.VMEM((CHUNK, D_FEAT), jnp.float32),
            pltpu.VMEM((CHUNK,), jnp.int32),
            pltpu.VMEM((CHUNK,), jnp.int32),
            pltpu.VMEM((CHUNK,), jnp.int32),
            pltpu.VMEM((TAIL,), jnp.int32),
            pltpu.VMEM((N_SEG, 16), jnp.float32),
            pltpu.VMEM_SHARED((N_SEG, D_FEAT), jnp.float32),
            pltpu.SemaphoreType.DMA,
            pltpu.SemaphoreType.DMA,
            pltpu.SemaphoreType.DMA,
            pltpu.SemaphoreType.DMA,
            pltpu.SemaphoreType.DMA,
            pltpu.SemaphoreType.DMA,
            pltpu.SemaphoreType.DMA,
            pltpu.SemaphoreType.DMA,
            pltpu.SemaphoreType.DMA,
        ],
    )
    partials, cnts = phase1(x, batch)
    out = pl.pallas_call(
        _combine,
        out_shape=jax.ShapeDtypeStruct((N_SEG, D_FEAT), jnp.float32),
    )(partials, cnts)
    return out


# speedup vs baseline: 1.0072x; 1.0072x over previous
import jax
import jax.numpy as jnp
from jax import lax
from jax.experimental import pallas as pl
from jax.experimental.pallas import tpu as pltpu
from jax.experimental.pallas import tpu_sc as plsc

N_NODES = 100000
D_FEAT = 128
N_SEG = 512
CHUNK = 128
NFULL = N_NODES // CHUNK
TAIL = N_NODES - NFULL * CHUNK
NC = 2
NS = 16
NW = NC * NS
CPW = (NFULL + NW - 1) // NW
NEXTRA = NFULL - (CPW - 1) * NW
NPAIR = (CPW + 1) // 2


def _phase1(x_hbm, b_hbm, part_hbm, cntp_hbm,
            xbA, xbB, ibA, ibB, tidx, cnt_local,
            acc_sh, semxA, semiA, semxB, semiB, semsA, semsB):
    cid = lax.axis_index("c")
    sid = lax.axis_index("s")
    w = sid * NC + cid

    cpw = jnp.where(w < NEXTRA, CPW, CPW - 1)
    base = (CPW - 1) * w + jnp.minimum(w, NEXTRA)

    def _init_cnt(i, _):
        cnt_local[i, :] = jnp.zeros((16,), jnp.float32)
        return 0
    lax.fori_loop(0, N_SEG, _init_cnt, 0)

    def _init_z(k, _):
        xbA[k // 8, pl.ds((k % 8) * 16, 16)] = jnp.zeros((16,), jnp.float32)
        return 0
    lax.fori_loop(0, 32 * 8, _init_z, 0)
    pltpu.sync_copy(xbA.at[pl.ds(0, 32)], acc_sh.at[pl.ds(sid * 32, 32)])

    def start_in(g, xb, ib, semx, semi):
        row0 = g * CHUNK
        pltpu.async_copy(b_hbm.at[pl.ds(row0, CHUNK)], ib, semi)
        pltpu.async_copy(x_hbm.at[pl.ds(row0, CHUNK)], xb, semx)

    def wait_in(xb, ib, semx, semi):
        pltpu.make_async_copy(b_hbm.at[pl.ds(0, CHUNK)], ib, semi).wait()
        pltpu.make_async_copy(x_hbm.at[pl.ds(0, CHUNK)], xb, semx).wait()

    ones16 = jnp.ones((16,), jnp.float32)

    def count_chunk(ib, ngroups):
        def body(j, _):
            v = ib[pl.ds(j * 16, 16)]
            for k in range(16):
                plsc.addupdate(cnt_local.at[v[k]], ones16)
            return 0
        lax.fori_loop(0, ngroups, body, 0)

    start_in(base, xbA, ibA, semxA, semiA)
    plsc.subcore_barrier()

    def _pair(c2, _):
        ke = 2 * c2
        ko = ke + 1
        kne = ke + 2

        @pl.when(ko < cpw)
        def _():
            start_in(base + ko, xbB, ibB, semxB, semiB)

        @pl.when(ke < cpw)
        def _():
            wait_in(xbA, ibA, semxA, semiA)
            pltpu.async_copy(xbA, acc_sh.at[ibA], semsA, add=True)
            count_chunk(ibA, CHUNK // 16)
            pltpu.make_async_copy(xbA, acc_sh.at[ibA], semsA).wait()

        @pl.when(kne < cpw)
        def _():
            start_in(base + kne, xbA, ibA, semxA, semiA)

        @pl.when(ko < cpw)
        def _():
            wait_in(xbB, ibB, semxB, semiB)
            pltpu.async_copy(xbB, acc_sh.at[ibB], semsB, add=True)
            count_chunk(ibB, CHUNK // 16)
            pltpu.make_async_copy(xbB, acc_sh.at[ibB], semsB).wait()
        return 0

    lax.fori_loop(0, NPAIR, _pair, 0)

    @pl.when(w == NW - 1)
    def _():
        row0 = NFULL * CHUNK
        pltpu.async_copy(b_hbm.at[pl.ds(row0, TAIL)], tidx, semiA).wait()
        pltpu.async_copy(x_hbm.at[pl.ds(row0, TAIL)],
                         xbA.at[pl.ds(0, TAIL)], semxA).wait()
        pltpu.async_copy(xbA.at[pl.ds(0, TAIL)], acc_sh.at[tidx],
                         semsA, add=True)
        count_chunk(tidx, TAIL // 16)
        pltpu.make_async_copy(xbA.at[pl.ds(0, TAIL)], acc_sh.at[tidx],
                              semsA).wait()

    pltpu.sync_copy(cnt_local, cntp_hbm.at[w])

    plsc.subcore_barrier()

    @pl.when(sid == 0)
    def _():
        pltpu.sync_copy(acc_sh, part_hbm.at[cid])


def _combine(p_ref, c_ref, o_ref):
    s = p_ref[0] + p_ref[1]
    cnt = jnp.sum(c_ref[...], axis=0)[:, 0:1]
    o_ref[...] = s / jnp.maximum(cnt, 1.0)


@jax.jit
def kernel(x, batch):
    mesh = plsc.VectorSubcoreMesh(core_axis_name="c", subcore_axis_name="s")
    phase1 = pl.kernel(
        _phase1,
        out_type=[
            jax.ShapeDtypeStruct((NC, N_SEG, D_FEAT), jnp.float32),
            jax.ShapeDtypeStruct((NW, N_SEG, 16), jnp.float32),
        ],
        mesh=mesh,
        scratch_types=[
            pltpu.VMEM((CHUNK, D_FEAT), jnp.float32),
            pltpu.VMEM((CHUNK, D_FEAT), jnp.float32),
            pltpu.VMEM((CHUNK,), jnp.int32),
            pltpu.VMEM((CHUNK,), jnp.int32),
            pltpu.VMEM((TAIL,), jnp.int32),
            pltpu.VMEM((N_SEG, 16), jnp.float32),
            pltpu.VMEM_SHARED((N_SEG, D_FEAT), jnp.float32),
            pltpu.SemaphoreType.DMA,
            pltpu.SemaphoreType.DMA,
            pltpu.SemaphoreType.DMA,
            pltpu.SemaphoreType.DMA,
            pltpu.SemaphoreType.DMA,
            pltpu.SemaphoreType.DMA,
        ],
    )
    partials, cnts = phase1(x, batch)
    out = pl.pallas_call(
        _combine,
        out_shape=jax.ShapeDtypeStruct((N_SEG, D_FEAT), jnp.float32),
    )(partials, cnts)
    return out


# speedup vs baseline: 1.0160x; 1.0088x over previous
import jax
import jax.numpy as jnp
from jax import lax
from jax.experimental import pallas as pl
from jax.experimental.pallas import tpu as pltpu
from jax.experimental.pallas import tpu_sc as plsc

N_NODES = 100000
D_FEAT = 128
N_SEG = 512
CHUNK = 128
NFULL = N_NODES // CHUNK
TAIL = N_NODES - NFULL * CHUNK
NC = 2
NS = 16
NW = NC * NS
CPW = (NFULL + NW - 1) // NW
NEXTRA = NFULL - (CPW - 1) * NW
NPAIR = (CPW + 1) // 2


def _phase1(x_hbm, b_hbm, part_hbm, cntp_hbm,
            xbA, xbB, ibA, ibB, tidx, cnt_local,
            acc_sh, semxA, semiA, semxB, semiB, semsA, semsB):
    cid = lax.axis_index("c")
    sid = lax.axis_index("s")
    w = sid * NC + cid

    cpw = jnp.where(w < NEXTRA, CPW, CPW - 1)
    base = (CPW - 1) * w + jnp.minimum(w, NEXTRA)

    def _init_cnt(i, _):
        cnt_local[i, :] = jnp.zeros((16,), jnp.float32)
        return 0
    lax.fori_loop(0, N_SEG, _init_cnt, 0)

    def _init_z(k, _):
        xbA[k // 8, pl.ds((k % 8) * 16, 16)] = jnp.zeros((16,), jnp.float32)
        return 0
    lax.fori_loop(0, 32 * 8, _init_z, 0)
    pltpu.sync_copy(xbA.at[pl.ds(0, 32)], acc_sh.at[pl.ds(sid * 32, 32)])

    H = CHUNK // 2

    def start_in(g, xb, ib, semx, semi):
        row0 = g * CHUNK
        pltpu.async_copy(b_hbm.at[pl.ds(row0, CHUNK)], ib, semi)
        pltpu.async_copy(x_hbm.at[pl.ds(row0, H)], xb.at[pl.ds(0, H)], semx)
        pltpu.async_copy(x_hbm.at[pl.ds(row0 + H, H)], xb.at[pl.ds(H, H)],
                         semi)

    def wait_in(xb, ib, semx, semi):
        pltpu.make_async_copy(b_hbm.at[pl.ds(0, CHUNK)], ib, semi).wait()
        pltpu.make_async_copy(x_hbm.at[pl.ds(0, H)], xb.at[pl.ds(0, H)],
                              semx).wait()
        pltpu.make_async_copy(x_hbm.at[pl.ds(0, H)], xb.at[pl.ds(H, H)],
                              semi).wait()

    ones16 = jnp.ones((16,), jnp.float32)

    def count_chunk(ib, ngroups):
        def body(j, _):
            v = ib[pl.ds(j * 16, 16)]
            for k in range(16):
                plsc.addupdate(cnt_local.at[v[k]], ones16)
            return 0
        lax.fori_loop(0, ngroups, body, 0)

    start_in(base, xbA, ibA, semxA, semiA)
    plsc.subcore_barrier()

    def _pair(c2, _):
        ke = 2 * c2
        ko = ke + 1
        kne = ke + 2

        @pl.when(ko < cpw)
        def _():
            start_in(base + ko, xbB, ibB, semxB, semiB)

        @pl.when(ke < cpw)
        def _():
            wait_in(xbA, ibA, semxA, semiA)
            pltpu.async_copy(xbA, acc_sh.at[ibA], semsA, add=True)
            count_chunk(ibA, CHUNK // 16)
            pltpu.make_async_copy(xbA, acc_sh.at[ibA], semsA).wait()

        @pl.when(kne < cpw)
        def _():
            start_in(base + kne, xbA, ibA, semxA, semiA)

        @pl.when(ko < cpw)
        def _():
            wait_in(xbB, ibB, semxB, semiB)
            pltpu.async_copy(xbB, acc_sh.at[ibB], semsB, add=True)
            count_chunk(ibB, CHUNK // 16)
            pltpu.make_async_copy(xbB, acc_sh.at[ibB], semsB).wait()
        return 0

    lax.fori_loop(0, NPAIR, _pair, 0)

    @pl.when(w == NW - 1)
    def _():
        row0 = NFULL * CHUNK
        pltpu.async_copy(b_hbm.at[pl.ds(row0, TAIL)], tidx, semiA).wait()
        pltpu.async_copy(x_hbm.at[pl.ds(row0, TAIL)],
                         xbA.at[pl.ds(0, TAIL)], semxA).wait()
        pltpu.async_copy(xbA.at[pl.ds(0, TAIL)], acc_sh.at[tidx],
                         semsA, add=True)
        count_chunk(tidx, TAIL // 16)
        pltpu.make_async_copy(xbA.at[pl.ds(0, TAIL)], acc_sh.at[tidx],
                              semsA).wait()

    pltpu.sync_copy(cnt_local, cntp_hbm.at[w])

    plsc.subcore_barrier()

    @pl.when(sid == 0)
    def _():
        pltpu.sync_copy(acc_sh, part_hbm.at[cid])


def _combine(p_ref, c_ref, o_ref):
    s = p_ref[0] + p_ref[1]
    cnt = jnp.sum(c_ref[...], axis=0)[:, 0:1]
    o_ref[...] = s / jnp.maximum(cnt, 1.0)


@jax.jit
def kernel(x, batch):
    mesh = plsc.VectorSubcoreMesh(core_axis_name="c", subcore_axis_name="s")
    phase1 = pl.kernel(
        _phase1,
        out_type=[
            jax.ShapeDtypeStruct((NC, N_SEG, D_FEAT), jnp.float32),
            jax.ShapeDtypeStruct((NW, N_SEG, 16), jnp.float32),
        ],
        mesh=mesh,
        scratch_types=[
            pltpu.VMEM((CHUNK, D_FEAT), jnp.float32),
            pltpu.VMEM((CHUNK, D_FEAT), jnp.float32),
            pltpu.VMEM((CHUNK,), jnp.int32),
            pltpu.VMEM((CHUNK,), jnp.int32),
            pltpu.VMEM((TAIL,), jnp.int32),
            pltpu.VMEM((N_SEG, 16), jnp.float32),
            pltpu.VMEM_SHARED((N_SEG, D_FEAT), jnp.float32),
            pltpu.SemaphoreType.DMA,
            pltpu.SemaphoreType.DMA,
            pltpu.SemaphoreType.DMA,
            pltpu.SemaphoreType.DMA,
            pltpu.SemaphoreType.DMA,
            pltpu.SemaphoreType.DMA,
        ],
    )
    partials, cnts = phase1(x, batch)
    out = pl.pallas_call(
        _combine,
        out_shape=jax.ShapeDtypeStruct((N_SEG, D_FEAT), jnp.float32),
    )(partials, cnts)
    return out


# speedup vs baseline: 1.0457x; 1.0292x over previous
import jax
import jax.numpy as jnp
from jax import lax
from jax.experimental import pallas as pl
from jax.experimental.pallas import tpu as pltpu
from jax.experimental.pallas import tpu_sc as plsc

N_NODES = 100000
D_FEAT = 128
N_SEG = 512
CHUNK = 128
NFULL = N_NODES // CHUNK
TAIL = N_NODES - NFULL * CHUNK
NC = 2
NS = 16
NW = NC * NS
CPW = (NFULL + NW - 1) // NW
NPAIR = (CPW + 1) // 2


def _phase1(x_hbm, b_hbm, part_hbm, cntp_hbm,
            xbA, xbB, ibA, ibB, tidx, cnt_local,
            acc_sh, semxA, semiA, semxB, semiB, semsA, semsB):
    cid = lax.axis_index("c")
    sid = lax.axis_index("s")
    w = sid * NC + cid

    def _init_cnt(i, _):
        cnt_local[i, :] = jnp.zeros((16,), jnp.float32)
        return 0
    lax.fori_loop(0, N_SEG, _init_cnt, 0)

    def _init_z(k, _):
        xbA[k // 8, pl.ds((k % 8) * 16, 16)] = jnp.zeros((16,), jnp.float32)
        return 0
    lax.fori_loop(0, 32 * 8, _init_z, 0)
    pltpu.sync_copy(xbA.at[pl.ds(0, 32)], acc_sh.at[pl.ds(sid * 32, 32)])

    def start_in(g, xb, ib, semx, semi):
        row0 = g * CHUNK
        pltpu.async_copy(b_hbm.at[pl.ds(row0, CHUNK)], ib, semi)
        pltpu.async_copy(x_hbm.at[pl.ds(row0, CHUNK)], xb, semx)

    def wait_in(xb, ib, semx, semi):
        pltpu.make_async_copy(b_hbm.at[pl.ds(0, CHUNK)], ib, semi).wait()
        pltpu.make_async_copy(x_hbm.at[pl.ds(0, CHUNK)], xb, semx).wait()

    ones16 = jnp.ones((16,), jnp.float32)

    def count_chunk(ib, ngroups):
        def body(j, _):
            v = ib[pl.ds(j * 16, 16)]
            for k in range(16):
                plsc.addupdate(cnt_local.at[v[k]], ones16)
            return 0
        lax.fori_loop(0, ngroups, body, 0)

    start_in(w, xbA, ibA, semxA, semiA)
    plsc.subcore_barrier()

    def _pair(c2, _):
        ge = w + NW * (2 * c2)
        go = ge + NW
        gne = ge + 2 * NW

        @pl.when(go < NFULL)
        def _():
            start_in(go, xbB, ibB, semxB, semiB)

        @pl.when(ge < NFULL)
        def _():
            wait_in(xbA, ibA, semxA, semiA)
            pltpu.async_copy(xbA, acc_sh.at[ibA], semsA, add=True)
            count_chunk(ibA, CHUNK // 16)
            pltpu.make_async_copy(xbA, acc_sh.at[ibA], semsA).wait()

        @pl.when(gne < NFULL)
        def _():
            start_in(gne, xbA, ibA, semxA, semiA)

        @pl.when(go < NFULL)
        def _():
            wait_in(xbB, ibB, semxB, semiB)
            pltpu.async_copy(xbB, acc_sh.at[ibB], semsB, add=True)
            count_chunk(ibB, CHUNK // 16)
            pltpu.make_async_copy(xbB, acc_sh.at[ibB], semsB).wait()
        return 0

    lax.fori_loop(0, NPAIR, _pair, 0)

    @pl.when(w == (NFULL % NW))
    def _():
        row0 = NFULL * CHUNK
        pltpu.async_copy(b_hbm.at[pl.ds(row0, TAIL)], tidx, semiA).wait()
        pltpu.async_copy(x_hbm.at[pl.ds(row0, TAIL)],
                         xbA.at[pl.ds(0, TAIL)], semxA).wait()
        pltpu.async_copy(xbA.at[pl.ds(0, TAIL)], acc_sh.at[tidx],
                         semsA, add=True)
        count_chunk(tidx, TAIL // 16)
        pltpu.make_async_copy(xbA.at[pl.ds(0, TAIL)], acc_sh.at[tidx],
                              semsA).wait()

    pltpu.sync_copy(cnt_local, cntp_hbm.at[w])

    plsc.subcore_barrier()

    @pl.when(sid == 0)
    def _():
        pltpu.sync_copy(acc_sh, part_hbm.at[cid])


def _combine(p_ref, c_ref, o_ref):
    s = p_ref[0] + p_ref[1]
    cnt = jnp.sum(c_ref[...], axis=0)[:, 0:1]
    o_ref[...] = s / jnp.maximum(cnt, 1.0)


@jax.jit
def kernel(x, batch):
    mesh = plsc.VectorSubcoreMesh(core_axis_name="c", subcore_axis_name="s")
    phase1 = pl.kernel(
        _phase1,
        out_type=[
            jax.ShapeDtypeStruct((NC, N_SEG, D_FEAT), jnp.float32),
            jax.ShapeDtypeStruct((NW, N_SEG, 16), jnp.float32),
        ],
        mesh=mesh,
        scratch_types=[
            pltpu.VMEM((CHUNK, D_FEAT), jnp.float32),
            pltpu.VMEM((CHUNK, D_FEAT), jnp.float32),
            pltpu.VMEM((CHUNK,), jnp.int32),
            pltpu.VMEM((CHUNK,), jnp.int32),
            pltpu.VMEM((TAIL,), jnp.int32),
            pltpu.VMEM((N_SEG, 16), jnp.float32),
            pltpu.VMEM_SHARED((N_SEG, D_FEAT), jnp.float32),
            pltpu.SemaphoreType.DMA,
            pltpu.SemaphoreType.DMA,
            pltpu.SemaphoreType.DMA,
            pltpu.SemaphoreType.DMA,
            pltpu.SemaphoreType.DMA,
            pltpu.SemaphoreType.DMA,
        ],
    )
    partials, cnts = phase1(x, batch)
    out = pl.pallas_call(
        _combine,
        out_shape=jax.ShapeDtypeStruct((N_SEG, D_FEAT), jnp.float32),
    )(partials, cnts)
    return out
